# baseline (device time: 33623 ns/iter reference)
import jax
import jax.numpy as jnp
from jax import lax
from jax.experimental import pallas as pl
from jax.experimental.pallas import tpu as pltpu

N_GLOBAL = 2048
EPS = 1e-5
BLK = 128
R = 512
DEPTH = 3


def kernel(x, gamma):
    m, n = x.shape
    nblk = m // BLK
    nb = m // R
    sub = R // BLK

    def body(x_hbm, g_ref, out_hbm, xv, send_ref, recv_ref,
             in_sems, out_sems, send_sem, recv_sem):
        my_x = lax.axis_index("x")
        my_y = lax.axis_index("y")
        nbr = (my_x, 1 - my_y)

        def in_copy(b):
            rows = pl.ds(b * R, R)
            return pltpu.make_async_copy(
                x_hbm.at[rows, :], xv.at[rows, :], in_sems.at[b % DEPTH]
            )

        def out_copy(b):
            rows = pl.ds(b * R, R)
            return pltpu.make_async_copy(
                xv.at[rows, :], out_hbm.at[rows, :], out_sems.at[b % DEPTH]
            )

        for b in range(min(DEPTH, nb)):
            in_copy(b).start()


        for b in range(nb):
            in_copy(b).wait()
            if b + DEPTH < nb:
                in_copy(b + DEPTH).start()
            for j in range(sub):
                i = b * sub + j
                xb = xv[pl.ds(i * BLK, BLK), :]
                send_ref[:, i : i + 1] = jnp.sum(xb * xb, axis=1, keepdims=True)

        rdma = pltpu.make_async_remote_copy(
            src_ref=send_ref,
            dst_ref=recv_ref,
            send_sem=send_sem,
            recv_sem=recv_sem,
            device_id=nbr,
            device_id_type=pl.DeviceIdType.MESH,
        )
        rdma.start()
        rdma.wait()

        total = send_ref[:, :] + recv_ref[:, :]
        scale = lax.rsqrt(total * (1.0 / N_GLOBAL) + EPS)
        gv = g_ref[:, :]

        for b in range(nb):
            for j in range(sub):
                i = b * sub + j
                rows = pl.ds(i * BLK, BLK)
                xv[rows, :] = xv[rows, :] * scale[:, i : i + 1] * gv
            if b >= DEPTH:
                out_copy(b - DEPTH).wait()
            out_copy(b).start()
        for b in range(max(nb - DEPTH, 0), nb):
            out_copy(b).wait()

    return pl.pallas_call(
        body,
        out_shape=jax.ShapeDtypeStruct((m, n), jnp.float32),
        in_specs=[
            pl.BlockSpec(memory_space=pl.ANY),
            pl.BlockSpec(memory_space=pltpu.VMEM),
        ],
        out_specs=pl.BlockSpec(memory_space=pl.ANY),
        scratch_shapes=[
            pltpu.VMEM((m, n), jnp.float32),
            pltpu.VMEM((BLK, nblk), jnp.float32),
            pltpu.VMEM((BLK, nblk), jnp.float32),
            pltpu.SemaphoreType.DMA((DEPTH,)),
            pltpu.SemaphoreType.DMA((DEPTH,)),
            pltpu.SemaphoreType.DMA,
            pltpu.SemaphoreType.DMA,
        ],
        compiler_params=pltpu.CompilerParams(
            vmem_limit_bytes=100 * 1024 * 1024,
        ),
    )(x, gamma.reshape(1, n))


# device time: 24781 ns/iter; 1.3568x vs baseline; 1.3568x over previous
import jax
import jax.numpy as jnp
from jax import lax
from jax.experimental import pallas as pl
from jax.experimental.pallas import tpu as pltpu

N_GLOBAL = 2048
EPS = 1e-5
BLK = 128
R = 512
DEPTH = 3


def _partials_kernel(x):
    m, n = x.shape
    nb = m // R
    sub = R // BLK

    def body(x_hbm, p_ref, xv, in_sems):
        def in_copy(b):
            rows = pl.ds(b * R, R)
            return pltpu.make_async_copy(
                x_hbm.at[rows, :], xv.at[rows, :], in_sems.at[b % DEPTH]
            )

        for b in range(min(DEPTH, nb)):
            in_copy(b).start()
        for b in range(nb):
            in_copy(b).wait()
            if b + DEPTH < nb:
                in_copy(b + DEPTH).start()
            for j in range(sub):
                i = b * sub + j
                xb = xv[pl.ds(i * BLK, BLK), :]
                p_ref[:, i : i + 1] = jnp.sum(xb * xb, axis=1, keepdims=True)

    return pl.pallas_call(
        body,
        out_shape=jax.ShapeDtypeStruct((BLK, m // BLK), jnp.float32),
        in_specs=[pl.BlockSpec(memory_space=pl.ANY)],
        out_specs=pl.BlockSpec(memory_space=pltpu.VMEM),
        scratch_shapes=[
            pltpu.VMEM((m, n), jnp.float32),
            pltpu.SemaphoreType.DMA((DEPTH,)),
        ],
        compiler_params=pltpu.CompilerParams(
            vmem_limit_bytes=100 * 1024 * 1024,
        ),
    )(x)


def _exchange_kernel(partial):

    def body(p_ref, s_ref, recv_ref, send_sem, recv_sem):
        my_x = lax.axis_index("x")
        my_y = lax.axis_index("y")
        nbr = (my_x, 1 - my_y)

        barrier_sem = pltpu.get_barrier_semaphore()
        pl.semaphore_signal(
            barrier_sem, inc=1, device_id=nbr,
            device_id_type=pl.DeviceIdType.MESH,
        )
        pl.semaphore_wait(barrier_sem, 1)

        rdma = pltpu.make_async_remote_copy(
            src_ref=p_ref,
            dst_ref=recv_ref,
            send_sem=send_sem,
            recv_sem=recv_sem,
            device_id=nbr,
            device_id_type=pl.DeviceIdType.MESH,
        )
        rdma.start()
        rdma.wait()

        total = p_ref[:, :] + recv_ref[:, :]
        s_ref[:, :] = lax.rsqrt(total * (1.0 / N_GLOBAL) + EPS)

    return pl.pallas_call(
        body,
        out_shape=jax.ShapeDtypeStruct(partial.shape, jnp.float32),
        in_specs=[pl.BlockSpec(memory_space=pltpu.VMEM)],
        out_specs=pl.BlockSpec(memory_space=pltpu.VMEM),
        scratch_shapes=[
            pltpu.VMEM(partial.shape, jnp.float32),
            pltpu.SemaphoreType.DMA,
            pltpu.SemaphoreType.DMA,
        ],
        compiler_params=pltpu.CompilerParams(
            collective_id=0,
            vmem_limit_bytes=100 * 1024 * 1024,
        ),
    )(partial)


def _normalize_kernel(x, scale, g2d):
    m, n = x.shape
    nb = m // R
    sub = R // BLK

    def body(x_hbm, s_ref, g_ref, out_hbm, xv, in_sems, out_sems):
        def in_copy(b):
            rows = pl.ds(b * R, R)
            return pltpu.make_async_copy(
                x_hbm.at[rows, :], xv.at[rows, :], in_sems.at[b % DEPTH]
            )

        def out_copy(b):
            rows = pl.ds(b * R, R)
            return pltpu.make_async_copy(
                xv.at[rows, :], out_hbm.at[rows, :], out_sems.at[b % DEPTH]
            )

        for b in range(min(DEPTH, nb)):
            in_copy(b).start()

        scale = s_ref[:, :]
        gv = g_ref[:, :]
        for b in range(nb):
            in_copy(b).wait()
            if b + DEPTH < nb:
                in_copy(b + DEPTH).start()
            for j in range(sub):
                i = b * sub + j
                rows = pl.ds(i * BLK, BLK)
                xv[rows, :] = xv[rows, :] * scale[:, i : i + 1] * gv
            if b >= DEPTH:
                out_copy(b - DEPTH).wait()
            out_copy(b).start()
        for b in range(max(nb - DEPTH, 0), nb):
            out_copy(b).wait()

    return pl.pallas_call(
        body,
        out_shape=jax.ShapeDtypeStruct((m, n), jnp.float32),
        in_specs=[
            pl.BlockSpec(memory_space=pl.ANY),
            pl.BlockSpec(memory_space=pltpu.VMEM),
            pl.BlockSpec(memory_space=pltpu.VMEM),
        ],
        out_specs=pl.BlockSpec(memory_space=pl.ANY),
        scratch_shapes=[
            pltpu.VMEM((m, n), jnp.float32),
            pltpu.SemaphoreType.DMA((DEPTH,)),
            pltpu.SemaphoreType.DMA((DEPTH,)),
        ],
        compiler_params=pltpu.CompilerParams(
            vmem_limit_bytes=100 * 1024 * 1024,
        ),
    )(x, scale, g2d)


def kernel(x, gamma):
    m, n = x.shape
    partial = _partials_kernel(x)
    scale = _exchange_kernel(partial)
    return _normalize_kernel(x, scale, gamma.reshape(1, n))


# device time: 22944 ns/iter; 1.4654x vs baseline; 1.0801x over previous
import jax
import jax.numpy as jnp
from jax import lax
from jax.experimental import pallas as pl
from jax.experimental.pallas import tpu as pltpu

N_GLOBAL = 2048
EPS = 1e-5
BLK = 128
R = 512
DEPTH = 4


def _scale_kernel(x):
    m, n = x.shape
    nb = m // R
    sub = R // BLK
    nblk = m // BLK
    half = nblk // 2

    def body(x_hbm, s_ref, xv, send_a, send_b, recv_a, recv_b, in_sems,
             send_sems, recv_sems):
        my_x = lax.axis_index("x")
        my_y = lax.axis_index("y")
        nbr = (my_x, 1 - my_y)

        def in_copy(b):
            rows = pl.ds(b * R, R)
            return pltpu.make_async_copy(
                x_hbm.at[rows, :], xv.at[rows, :], in_sems.at[b % DEPTH]
            )

        def rdma_half(h):
            src, dst = (send_a, recv_a) if h == 0 else (send_b, recv_b)
            return pltpu.make_async_remote_copy(
                src_ref=src,
                dst_ref=dst,
                send_sem=send_sems.at[h],
                recv_sem=recv_sems.at[h],
                device_id=nbr,
                device_id_type=pl.DeviceIdType.MESH,
            )

        for b in range(min(DEPTH, nb)):
            in_copy(b).start()

        barrier_sem = pltpu.get_barrier_semaphore()
        pl.semaphore_signal(
            barrier_sem, inc=1, device_id=nbr,
            device_id_type=pl.DeviceIdType.MESH,
        )
        pl.semaphore_wait(barrier_sem, 1)

        for b in range(nb):
            in_copy(b).wait()
            if b + DEPTH < nb:
                in_copy(b + DEPTH).start()
            for j in range(sub):
                i = b * sub + j
                xb = xv[pl.ds(i * BLK, BLK), :]
                ps = jnp.sum(xb * xb, axis=1, keepdims=True)
                if i < half:
                    send_a[:, i : i + 1] = ps
                else:
                    send_b[:, i - half : i - half + 1] = ps
            if b == nb // 2 - 1:
                rdma_half(0).start()
        rdma_half(1).start()
        rdma_half(0).wait()
        rdma_half(1).wait()

        tot_a = send_a[:, :] + recv_a[:, :]
        tot_b = send_b[:, :] + recv_b[:, :]
        s_ref[:, :half] = lax.rsqrt(tot_a * (1.0 / N_GLOBAL) + EPS)
        s_ref[:, half:] = lax.rsqrt(tot_b * (1.0 / N_GLOBAL) + EPS)

    return pl.pallas_call(
        body,
        out_shape=jax.ShapeDtypeStruct((BLK, nblk), jnp.float32),
        in_specs=[pl.BlockSpec(memory_space=pl.ANY)],
        out_specs=pl.BlockSpec(memory_space=pltpu.VMEM),
        scratch_shapes=[
            pltpu.VMEM((m, n), jnp.float32),
            pltpu.VMEM((BLK, half), jnp.float32),
            pltpu.VMEM((BLK, half), jnp.float32),
            pltpu.VMEM((BLK, half), jnp.float32),
            pltpu.VMEM((BLK, half), jnp.float32),
            pltpu.SemaphoreType.DMA((DEPTH,)),
            pltpu.SemaphoreType.DMA((2,)),
            pltpu.SemaphoreType.DMA((2,)),
        ],
        compiler_params=pltpu.CompilerParams(
            collective_id=0,
            vmem_limit_bytes=100 * 1024 * 1024,
        ),
    )(x)


def _normalize_kernel(x, scale, g2d):
    m, n = x.shape
    nb = m // R
    sub = R // BLK

    def body(x_hbm, s_ref, g_ref, out_hbm, xv, in_sems, out_sems):
        def in_copy(b):
            rows = pl.ds(b * R, R)
            return pltpu.make_async_copy(
                x_hbm.at[rows, :], xv.at[rows, :], in_sems.at[b % DEPTH]
            )

        def out_copy(b):
            rows = pl.ds(b * R, R)
            return pltpu.make_async_copy(
                xv.at[rows, :], out_hbm.at[rows, :], out_sems.at[b % DEPTH]
            )

        for b in range(min(DEPTH, nb)):
            in_copy(b).start()

        scale = s_ref[:, :]
        gv = g_ref[:, :]
        for b in range(nb):
            in_copy(b).wait()
            if b + DEPTH < nb:
                in_copy(b + DEPTH).start()
            for j in range(sub):
                i = b * sub + j
                rows = pl.ds(i * BLK, BLK)
                xv[rows, :] = xv[rows, :] * scale[:, i : i + 1] * gv
            if b >= DEPTH:
                out_copy(b - DEPTH).wait()
            out_copy(b).start()
        for b in range(max(nb - DEPTH, 0), nb):
            out_copy(b).wait()

    return pl.pallas_call(
        body,
        out_shape=jax.ShapeDtypeStruct((m, n), jnp.float32),
        in_specs=[
            pl.BlockSpec(memory_space=pl.ANY),
            pl.BlockSpec(memory_space=pltpu.VMEM),
            pl.BlockSpec(memory_space=pltpu.VMEM),
        ],
        out_specs=pl.BlockSpec(memory_space=pl.ANY),
        scratch_shapes=[
            pltpu.VMEM((m, n), jnp.float32),
            pltpu.SemaphoreType.DMA((DEPTH,)),
            pltpu.SemaphoreType.DMA((DEPTH,)),
        ],
        compiler_params=pltpu.CompilerParams(
            vmem_limit_bytes=100 * 1024 * 1024,
        ),
    )(x, scale, g2d)


def kernel(x, gamma):
    m, n = x.shape
    scale = _scale_kernel(x)
    return _normalize_kernel(x, scale, gamma.reshape(1, n))
